# baseline (speedup 1.0000x reference)
import jax
import jax.numpy as jnp
from jax import lax
from jax.experimental import pallas as pl
from jax.experimental.pallas import tpu as pltpu
from jax.experimental.pallas import tpu_sc as plsc

EMBED = 32
BATCH = 16384
NUM_WORKERS = 32
B_PER_W = BATCH // NUM_WORKERS


def _gather_body(uidx_hbm, midx_hbm, utab_hbm, mtab_hbm, uout_hbm, mout_hbm,
                 idx_u, idx_m, rows_u, rows_m, sem_u, sem_m):
    wid = lax.axis_index("s") * 2 + lax.axis_index("c")
    base = wid * B_PER_W
    pltpu.sync_copy(uidx_hbm.at[pl.ds(base, B_PER_W)], idx_u)
    pltpu.sync_copy(midx_hbm.at[pl.ds(base, B_PER_W)], idx_m)
    cu = pltpu.async_copy(utab_hbm.at[idx_u], rows_u, sem_u)
    cm = pltpu.async_copy(mtab_hbm.at[idx_m], rows_m, sem_m)
    cu.wait()
    cm.wait()
    pltpu.sync_copy(rows_u, uout_hbm.at[pl.ds(base, B_PER_W)])
    pltpu.sync_copy(rows_m, mout_hbm.at[pl.ds(base, B_PER_W)])


def _mlp_body(u_ref, m_ref, w1a_ref, w1b_ref, b1_ref, w2_ref, b2_ref, o_ref):
    h = (jnp.dot(u_ref[...], w1a_ref[...], preferred_element_type=jnp.float32)
         + jnp.dot(m_ref[...], w1b_ref[...], preferred_element_type=jnp.float32)
         + b1_ref[...])
    h = jnp.maximum(h, 0.0)
    o = jnp.dot(h, w2_ref[...], preferred_element_type=jnp.float32) + b2_ref[...]
    o_ref[...] = 5.0 * jax.nn.sigmoid(o)


def kernel(inputs, user_embedding, movie_embedding, W1, b1, W2, b2):
    uidx = inputs[:, 0]
    midx = inputs[:, 1]

    mesh = plsc.VectorSubcoreMesh(core_axis_name="c", subcore_axis_name="s")
    u_rows, m_rows = pl.kernel(
        _gather_body,
        mesh=mesh,
        out_type=[
            jax.ShapeDtypeStruct((BATCH, EMBED), jnp.float32),
            jax.ShapeDtypeStruct((BATCH, EMBED), jnp.float32),
        ],
        scratch_types=[
            pltpu.VMEM((B_PER_W,), jnp.int32),
            pltpu.VMEM((B_PER_W,), jnp.int32),
            pltpu.VMEM((B_PER_W, EMBED), jnp.float32),
            pltpu.VMEM((B_PER_W, EMBED), jnp.float32),
            pltpu.SemaphoreType.DMA,
            pltpu.SemaphoreType.DMA,
        ],
        compiler_params=pltpu.CompilerParams(use_tc_tiling_on_sc=False),
    )(uidx, midx, user_embedding, movie_embedding)

    BT = 2048
    out = pl.pallas_call(
        _mlp_body,
        grid=(BATCH // BT,),
        in_specs=[
            pl.BlockSpec((BT, EMBED), lambda i: (i, 0)),
            pl.BlockSpec((BT, EMBED), lambda i: (i, 0)),
            pl.BlockSpec((EMBED, EMBED), lambda i: (0, 0)),
            pl.BlockSpec((EMBED, EMBED), lambda i: (0, 0)),
            pl.BlockSpec((1, EMBED), lambda i: (0, 0)),
            pl.BlockSpec((EMBED, 1), lambda i: (0, 0)),
            pl.BlockSpec((1, 1), lambda i: (0, 0)),
        ],
        out_specs=pl.BlockSpec((BT, 1), lambda i: (i, 0)),
        out_shape=jax.ShapeDtypeStruct((BATCH, 1), jnp.float32),
    )(u_rows, m_rows, W1[:EMBED], W1[EMBED:], b1.reshape(1, EMBED),
      W2, b2.reshape(1, 1))
    return out.reshape(-1)
